# baseline (speedup 1.0000x reference)
import functools

import jax
import jax.numpy as jnp
from jax import lax
from jax.experimental import pallas as pl
from jax.experimental.pallas import tpu as pltpu
from jax.experimental.pallas import tpu_sc as plsc

H = 64
W = 64
HW = H * W
B = 8
NPIX = B * HW
NW = 32
PIX_PER_W = NPIX // NW
K = 4
GATH_PER_W = K * PIX_PER_W
CHUNK = 128
NCHUNK = GATH_PER_W // CHUNK
NVEC = PIX_PER_W // 16
TOTAL = B * K * HW


def _sc_body(corr_hbm, gx_hbm, gy_hbm, vis_hbm, out_hbm,
             gx_v, gy_v, vis_v, idx_v, w_v, vals_v, acc_v, sem):
    wid = lax.axis_index("s") * 2 + lax.axis_index("c")
    base = wid * PIX_PER_W

    pltpu.sync_copy(gx_hbm.at[pl.ds(base, PIX_PER_W)], gx_v)
    pltpu.sync_copy(gy_hbm.at[pl.ds(base, PIX_PER_W)], gy_v)
    pltpu.sync_copy(vis_hbm.at[pl.ds(base, PIX_PER_W)], vis_v)

    lane = lax.iota(jnp.int32, 16)

    def compute(i, _):
        off = i * 16
        gx = gx_v[pl.ds(off, 16)]
        gy = gy_v[pl.ds(off, 16)]
        vis = vis_v[pl.ds(off, 16)]
        gxg = (gx + 1.0) * ((W - 1.0) / 2.0)
        gyg = (gy + 1.0) * ((H - 1.0) / 2.0)
        fxi = gxg.astype(jnp.int32)
        fyi = gyg.astype(jnp.int32)
        fx = fxi.astype(jnp.float32)
        fy = fyi.astype(jnp.float32)
        wx0 = fx + 1.0 - gxg
        wx1 = gxg - fx
        wy0 = fy + 1.0 - gyg
        wy1 = gyg - fy
        rowbase = (base + off + lane) * HW
        for k, (ox, oy, wgt) in enumerate((
                (0, 0, wy0 * wx0),
                (0, 1, wy0 * wx1),
                (1, 0, wy1 * wx0),
                (1, 1, wy1 * wx1))):
            xi = jnp.minimum(jnp.maximum(fyi + ox, 0), H - 1)
            yi = jnp.minimum(jnp.maximum(fxi + oy, 0), W - 1)
            pos = k * PIX_PER_W + off
            idx_v[pl.ds(pos, 16)] = rowbase + xi * W + yi
            w_v[pl.ds(pos, 16)] = wgt * vis
        return 0

    lax.fori_loop(0, NVEC, compute, 0)

    def fire(c, _):
        s = c * CHUNK
        pltpu.async_copy(corr_hbm.at[idx_v.at[pl.ds(s, CHUNK)]],
                         vals_v.at[pl.ds(s, CHUNK)], sem)
        return 0

    lax.fori_loop(0, NCHUNK, fire, 0)
    pltpu.make_async_copy(corr_hbm.at[pl.ds(0, GATH_PER_W)], vals_v, sem).wait()

    def reduce(j, acc):
        off = j * 16
        pixoff = (j % NVEC) * 16
        v = vals_v[pl.ds(off, 16)]
        w = w_v[pl.ds(off, 16)]
        vis = vis_v[pl.ds(pixoff, 16)]
        return acc + jnp.abs(v * vis - w)

    acc = lax.fori_loop(0, K * NVEC, reduce,
                        jnp.zeros((16,), jnp.float32))
    acc_v[...] = acc * (1.0 / TOTAL)
    pltpu.sync_copy(acc_v, out_hbm.at[wid])


@jax.jit
def kernel(corr_m, gt_flow, vis_mask):
    corr_flat = corr_m.reshape(-1)
    gx = gt_flow[:, 0].reshape(-1)
    gy = gt_flow[:, 1].reshape(-1)
    vis = vis_mask.reshape(-1)

    mesh = plsc.VectorSubcoreMesh(core_axis_name="c", subcore_axis_name="s")
    out = pl.kernel(
        _sc_body,
        out_type=jax.ShapeDtypeStruct((NW, 16), jnp.float32),
        mesh=mesh,
        scratch_types=[
            pltpu.VMEM((PIX_PER_W,), jnp.float32),
            pltpu.VMEM((PIX_PER_W,), jnp.float32),
            pltpu.VMEM((PIX_PER_W,), jnp.float32),
            pltpu.VMEM((GATH_PER_W,), jnp.int32),
            pltpu.VMEM((GATH_PER_W,), jnp.float32),
            pltpu.VMEM((GATH_PER_W,), jnp.float32),
            pltpu.VMEM((16,), jnp.float32),
            pltpu.SemaphoreType.DMA,
        ],
    )(corr_flat, gx, gy, vis)
    return jnp.sum(out)


# speedup vs baseline: 12.9599x; 12.9599x over previous
import functools

import jax
import jax.numpy as jnp
from jax import lax
from jax.experimental import pallas as pl
from jax.experimental.pallas import tpu as pltpu
from jax.experimental.pallas import tpu_sc as plsc

H = 64
W = 64
HW = H * W
B = 8
NPIX = B * HW
NW = 32
PIX_PER_W = NPIX // NW
K = 4
GATH_PER_W = K * PIX_PER_W
CHUNK = 128
NCHUNK = GATH_PER_W // CHUNK
NVEC = PIX_PER_W // 16
TOTAL = B * K * HW


def _sc_body(corr_hbm, gx_hbm, gy_hbm, vis_hbm, out_hbm,
             gx_v, gy_v, vis_v, idx_v, w_v, vals_v, acc_v, sem):
    wid = lax.axis_index("s") * 2 + lax.axis_index("c")
    base = wid * PIX_PER_W

    pltpu.sync_copy(gx_hbm.at[pl.ds(base, PIX_PER_W)], gx_v)
    pltpu.sync_copy(gy_hbm.at[pl.ds(base, PIX_PER_W)], gy_v)
    pltpu.sync_copy(vis_hbm.at[pl.ds(base, PIX_PER_W)], vis_v)

    lane = lax.iota(jnp.int32, 16)

    def compute(i, _):
        off = i * 16
        gx = gx_v[pl.ds(off, 16)]
        gy = gy_v[pl.ds(off, 16)]
        vis = vis_v[pl.ds(off, 16)]
        gxg = (gx + 1.0) * ((W - 1.0) / 2.0)
        gyg = (gy + 1.0) * ((H - 1.0) / 2.0)
        fxi = gxg.astype(jnp.int32)
        fyi = gyg.astype(jnp.int32)
        fx = fxi.astype(jnp.float32)
        fy = fyi.astype(jnp.float32)
        wx0 = fx + 1.0 - gxg
        wx1 = gxg - fx
        wy0 = fy + 1.0 - gyg
        wy1 = gyg - fy
        m = base + off + lane
        bb = m >> 12
        r = m & (HW - 1)
        rowpart = (bb << 24) + ((r >> 3) << 15) + ((r & 7) << 7)
        for k, (ox, oy, wgt) in enumerate((
                (0, 0, wy0 * wx0),
                (0, 1, wy0 * wx1),
                (1, 0, wy1 * wx0),
                (1, 1, wy1 * wx1))):
            xi = jnp.minimum(jnp.maximum(fyi + ox, 0), H - 1)
            yi = jnp.minimum(jnp.maximum(fxi + oy, 0), W - 1)
            c = xi * W + yi
            pos = k * PIX_PER_W + off
            idx_v[pl.ds(pos, 16)] = rowpart + ((c >> 7) << 10) + (c & 127)
            w_v[pl.ds(pos, 16)] = wgt * vis
        return 0

    lax.fori_loop(0, NVEC, compute, 0)

    def fire(c, _):
        s = c * CHUNK
        pltpu.async_copy(corr_hbm.at[idx_v.at[pl.ds(s, CHUNK)]],
                         vals_v.at[pl.ds(s, CHUNK)], sem)
        return 0

    lax.fori_loop(0, NCHUNK, fire, 0)
    pltpu.make_async_copy(corr_hbm.at[pl.ds(0, GATH_PER_W)], vals_v, sem).wait()

    def reduce(j, acc):
        off = j * 16
        pixoff = (j % NVEC) * 16
        v = vals_v[pl.ds(off, 16)]
        w = w_v[pl.ds(off, 16)]
        vis = vis_v[pl.ds(pixoff, 16)]
        return acc + jnp.abs(v * vis - w)

    acc = lax.fori_loop(0, K * NVEC, reduce,
                        jnp.zeros((16,), jnp.float32))
    acc_v[...] = acc * (1.0 / TOTAL)
    pltpu.sync_copy(acc_v, out_hbm.at[wid])


@jax.jit
def kernel(corr_m, gt_flow, vis_mask):
    corr_flat = (
        corr_m.reshape(B, HW // 8, 8, HW // 128, 128)
        .transpose(0, 1, 3, 2, 4)
        .reshape(-1)
    )
    gx = gt_flow[:, 0].reshape(-1)
    gy = gt_flow[:, 1].reshape(-1)
    vis = vis_mask.reshape(-1)

    mesh = plsc.VectorSubcoreMesh(core_axis_name="c", subcore_axis_name="s")
    out = pl.kernel(
        _sc_body,
        out_type=jax.ShapeDtypeStruct((NW, 16), jnp.float32),
        mesh=mesh,
        scratch_types=[
            pltpu.VMEM((PIX_PER_W,), jnp.float32),
            pltpu.VMEM((PIX_PER_W,), jnp.float32),
            pltpu.VMEM((PIX_PER_W,), jnp.float32),
            pltpu.VMEM((GATH_PER_W,), jnp.int32),
            pltpu.VMEM((GATH_PER_W,), jnp.float32),
            pltpu.VMEM((GATH_PER_W,), jnp.float32),
            pltpu.VMEM((16,), jnp.float32),
            pltpu.SemaphoreType.DMA,
        ],
    )(corr_flat, gx, gy, vis)
    return jnp.sum(out)


# speedup vs baseline: 13.4795x; 1.0401x over previous
import functools

import jax
import jax.numpy as jnp
from jax import lax
from jax.experimental import pallas as pl
from jax.experimental.pallas import tpu as pltpu
from jax.experimental.pallas import tpu_sc as plsc

H = 64
W = 64
HW = H * W
B = 8
NPIX = B * HW
NW = 32
PIX_PER_W = NPIX // NW
K = 4
GATH_PER_W = K * PIX_PER_W
CHUNK = 128
NCHUNK = GATH_PER_W // CHUNK
NVEC = PIX_PER_W // 16
TOTAL = B * K * HW


def _sc_body(corr_hbm, gx_hbm, gy_hbm, vis_hbm, out_hbm,
             gx_v, gy_v, vis_v, idx_v, w_v, vals_v, acc_v, sem):
    wid = lax.axis_index("s") * 2 + lax.axis_index("c")
    base = wid * PIX_PER_W

    pltpu.sync_copy(gx_hbm.at[pl.ds(base, PIX_PER_W)], gx_v)
    pltpu.sync_copy(gy_hbm.at[pl.ds(base, PIX_PER_W)], gy_v)
    pltpu.sync_copy(vis_hbm.at[pl.ds(base, PIX_PER_W)], vis_v)

    lane = lax.iota(jnp.int32, 16)

    def block(blk, _):
        blkoff = blk * CHUNK
        for ii in range(CHUNK // 16):
            off = blkoff + ii * 16
            gx = gx_v[pl.ds(off, 16)]
            gy = gy_v[pl.ds(off, 16)]
            vis = vis_v[pl.ds(off, 16)]
            gxg = (gx + 1.0) * ((W - 1.0) / 2.0)
            gyg = (gy + 1.0) * ((H - 1.0) / 2.0)
            fxi = gxg.astype(jnp.int32)
            fyi = gyg.astype(jnp.int32)
            fx = fxi.astype(jnp.float32)
            fy = fyi.astype(jnp.float32)
            wx0 = fx + 1.0 - gxg
            wx1 = gxg - fx
            wy0 = fy + 1.0 - gyg
            wy1 = gyg - fy
            m = base + off + lane
            bb = m >> 12
            r = m & (HW - 1)
            rowpart = (bb << 24) + ((r >> 3) << 15) + ((r & 7) << 7)
            for k, (ox, oy, wgt) in enumerate((
                    (0, 0, wy0 * wx0),
                    (0, 1, wy0 * wx1),
                    (1, 0, wy1 * wx0),
                    (1, 1, wy1 * wx1))):
                xi = jnp.minimum(jnp.maximum(fyi + ox, 0), H - 1)
                yi = jnp.minimum(jnp.maximum(fxi + oy, 0), W - 1)
                c = xi * W + yi
                pos = k * PIX_PER_W + off
                idx_v[pl.ds(pos, 16)] = rowpart + ((c >> 7) << 10) + (c & 127)
                w_v[pl.ds(pos, 16)] = wgt * vis
        for k in range(K):
            s = k * PIX_PER_W + blkoff
            pltpu.async_copy(corr_hbm.at[idx_v.at[pl.ds(s, CHUNK)]],
                             vals_v.at[pl.ds(s, CHUNK)], sem)
        return 0

    lax.fori_loop(0, PIX_PER_W // CHUNK, block, 0)
    pltpu.make_async_copy(corr_hbm.at[pl.ds(0, GATH_PER_W)], vals_v, sem).wait()

    def reduce(j, acc):
        off = j * 16
        vis = vis_v[pl.ds(off, 16)]
        for k in range(K):
            pos = k * PIX_PER_W + off
            acc = acc + jnp.abs(vals_v[pl.ds(pos, 16)] * vis
                                - w_v[pl.ds(pos, 16)])
        return acc

    acc = lax.fori_loop(0, NVEC, reduce,
                        jnp.zeros((16,), jnp.float32), unroll=2)
    acc_v[...] = acc * (1.0 / TOTAL)
    pltpu.sync_copy(acc_v, out_hbm.at[wid])


@jax.jit
def kernel(corr_m, gt_flow, vis_mask):
    corr_flat = (
        corr_m.reshape(B, HW // 8, 8, HW // 128, 128)
        .transpose(0, 1, 3, 2, 4)
        .reshape(-1)
    )
    gx = gt_flow[:, 0].reshape(-1)
    gy = gt_flow[:, 1].reshape(-1)
    vis = vis_mask.reshape(-1)

    mesh = plsc.VectorSubcoreMesh(core_axis_name="c", subcore_axis_name="s")
    out = pl.kernel(
        _sc_body,
        out_type=jax.ShapeDtypeStruct((NW, 16), jnp.float32),
        mesh=mesh,
        scratch_types=[
            pltpu.VMEM((PIX_PER_W,), jnp.float32),
            pltpu.VMEM((PIX_PER_W,), jnp.float32),
            pltpu.VMEM((PIX_PER_W,), jnp.float32),
            pltpu.VMEM((GATH_PER_W,), jnp.int32),
            pltpu.VMEM((GATH_PER_W,), jnp.float32),
            pltpu.VMEM((GATH_PER_W,), jnp.float32),
            pltpu.VMEM((16,), jnp.float32),
            pltpu.SemaphoreType.DMA,
        ],
    )(corr_flat, gx, gy, vis)
    return jnp.sum(out)
